# baseline (speedup 1.0000x reference)
import functools
import jax
import jax.numpy as jnp
from jax import lax
from jax.experimental import pallas as pl
from jax.experimental.pallas import tpu as pltpu

N_NODES = 10000
N_EDGES = 160000
H = 128
MID = 64
CUTOFF = 8.0

EB = 1600
NB = 1000


def _sig(x):
    return 1.0 / (1.0 + jnp.exp(-x))


def _sw(x):
    return x * _sig(x)


def _cross(u, v):
    ux, uy, uz = u[:, 0:1], u[:, 1:2], u[:, 2:3]
    vx, vy, vz = v[:, 0:1], v[:, 1:2], v[:, 2:3]
    return jnp.concatenate(
        [uy * vz - uz * vy, uz * vx - ux * vz, ux * vy - uy * vx], axis=1)


def _dot3(u, v):
    return jnp.sum(u * v, axis=1, keepdims=True)


def _atan2pos(b, a):
    t = jnp.arctan2(b, a)
    return jnp.where(t < 0, t + jnp.pi, t)


def _edge_body(geo_ref, bond_ref, xj_ref,
               g1_w, g1_b, g2_w, g2_b, bw, bb,
               el1_w, el1_b, el2a_w, el2a_b, el2b_w, el2b_b,
               at1_w, at1_b, at2_w, at2_b,
               m1_ref, m2_ref):
    g = geo_ref[...]
    pji = g[:, 0:3]
    pin0 = g[:, 3:6]
    pin1 = g[:, 6:9]
    piref = g[:, 9:12]
    pjref = g[:, 12:15]

    npji = -pji
    a_t = _dot3(npji, pin0)
    c_t = _cross(npji, pin0)
    b_t = jnp.sqrt(_dot3(c_t, c_t) + 1e-12)
    theta = _atan2pos(b_t, a_t)

    dist = jnp.sqrt(_dot3(pji, pji) + 1e-12)

    p1 = c_t
    p2 = _cross(npji, pin1)
    a_p = _dot3(p1, p2)
    b_p = _dot3(_cross(p1, p2), pji) / dist
    phi = _atan2pos(b_p, a_p)

    q1 = _cross(pji, pjref)
    q2 = _cross(pji, piref)
    a_u = _dot3(q1, q2)
    b_u = _dot3(_cross(q1, q2), pji) / dist
    tau = _atan2pos(b_u, a_u)

    geom_in = jnp.concatenate([dist, theta, phi, tau], axis=1)
    gm = _sw(jnp.dot(geom_in, g1_w[...]) + g1_b[...])
    gm = jnp.dot(gm, g2_w[...]) + g2_b[...]
    bd = _sw(jnp.dot(bond_ref[...], bw[...]) + bb[...])

    ew = jnp.concatenate([bd, gm], axis=1)
    e1 = _sw(jnp.dot(ew, el1_w[...]) + el1_b[...])
    ew1 = jnp.dot(e1[:, :H], el2a_w[...]) + el2a_b[...]
    ew2 = jnp.dot(e1[:, H:], el2b_w[...]) + el2b_b[...]

    xj = xj_ref[...]
    m1 = ew1 * xj
    m1 = m1 * _sig(jnp.sum(m1 * at1_w[...], axis=1, keepdims=True) + at1_b[...])
    m2 = ew2 * xj
    m2 = m2 * _sig(jnp.sum(m2 * at2_w[...], axis=1, keepdims=True) + at2_b[...])
    m1_ref[...] = m1
    m2_ref[...] = m2


def _node_body(a1_ref, a2_ref, x_ref,
               rel1_w, rel1_b, root1_w, rel2_w, rel2_b, root2_w,
               lin1_w, lin1_b, lin2_w, lin2_b, cat_w, cat_b,
               h_ref, st_ref):
    x = x_ref[...]
    h1 = jnp.dot(a1_ref[...], rel1_w[...]) + rel1_b[...] + jnp.dot(x, root1_w[...])
    h1 = _sw(jnp.dot(h1, lin1_w[...]) + lin1_b[...])
    h2 = jnp.dot(a2_ref[...], rel2_w[...]) + rel2_b[...] + jnp.dot(x, root2_w[...])
    h2 = _sw(jnp.dot(h2, lin2_w[...]) + lin2_b[...])
    hc = jnp.concatenate([h1, h2], axis=1)
    h = jnp.dot(hc, cat_w[...]) + cat_b[...] + x
    h_ref[...] = h

    @pl.when(pl.program_id(0) == 0)
    def _():
        st_ref[...] = jnp.zeros_like(st_ref)

    st_ref[...] += jnp.concatenate(
        [jnp.sum(h, axis=0, keepdims=True),
         jnp.sum(h * h, axis=0, keepdims=True)], axis=0)


def _final_body(h_ref, st_ref, gn_g, gn_b, mlp_w, mlp_b, fin_w, fin_b, o_ref):
    st = st_ref[...]
    mean = st[0:1, :] / N_NODES
    var = st[1:2, :] / N_NODES - mean * mean
    h = (h_ref[...] - mean) / jnp.sqrt(var + 1e-5) * gn_g[...] + gn_b[...]
    h = _sw(jnp.dot(h, mlp_w[...]) + mlp_b[...])
    o_ref[...] = jnp.dot(h, fin_w[...]) + fin_b[...]


def _rep(shape):
    return pl.BlockSpec(shape, lambda b: (0,) * len(shape))


def _seg_min_arg(vals, seg, num_segments):
    e = vals.shape[0]
    minv = jax.ops.segment_min(vals, seg, num_segments=num_segments)
    cand = jnp.where(vals <= minv[seg], jnp.arange(e), e)
    arg = jax.ops.segment_min(cand, seg, num_segments=num_segments)
    return jnp.where(arg >= e, 0, arg)


def kernel(z, pos, edge_index, edge_bond_attr, params):
    p = params
    i = edge_index[0]
    j = edge_index[1]
    vecs = pos[j] - pos[i]
    dist = jnp.sqrt(jnp.sum(vecs * vecs, axis=-1) + 1e-12)

    am0 = _seg_min_arg(dist, i, N_NODES)
    d1 = dist.at[am0].set(jnp.full((N_NODES,), CUTOFF) + dist[am0])
    am1 = _seg_min_arg(d1, i, N_NODES)
    am0j = _seg_min_arg(dist, j, N_NODES)
    d1j = dist.at[am0j].set(jnp.full((N_NODES,), CUTOFF) + dist[am0j])
    am1j = _seg_min_arg(d1j, j, N_NODES)

    n0 = j[am0]
    n0j = i[am0j]
    am0i_e = am0[i]
    am1i_e = am1[i]
    idx_iref = jnp.where(n0[i] == j, am1i_e, am0i_e)
    idx_jref = jnp.where(n0j[j] == i, am1j[j], am0j[j])

    geo = jnp.concatenate(
        [vecs, vecs[am0i_e], vecs[am1i_e], vecs[idx_iref], vecs[idx_jref],
         jnp.zeros((N_EDGES, 1), jnp.float32)], axis=1)

    x = _sw(p['emb'][z])
    xj = x[j]

    el1_w = jnp.concatenate([p['c1_el1_w'], p['c2_el1_w']], axis=1)
    el1_b = jnp.concatenate([p['c1_el1_b'], p['c2_el1_b']])[None, :]

    m1, m2 = pl.pallas_call(
        _edge_body,
        grid=(N_EDGES // EB,),
        in_specs=[
            pl.BlockSpec((EB, 16), lambda b: (b, 0)),
            pl.BlockSpec((EB, 16), lambda b: (b, 0)),
            pl.BlockSpec((EB, H), lambda b: (b, 0)),
            _rep((4, MID)), _rep((1, MID)), _rep((MID, H)), _rep((1, H)),
            _rep((16, H)), _rep((1, H)),
            _rep((2 * H, 2 * H)), _rep((1, 2 * H)),
            _rep((H, H)), _rep((1, H)), _rep((H, H)), _rep((1, H)),
            _rep((1, H)), _rep((1, 1)), _rep((1, H)), _rep((1, 1)),
        ],
        out_specs=[pl.BlockSpec((EB, H), lambda b: (b, 0)),
                   pl.BlockSpec((EB, H), lambda b: (b, 0))],
        out_shape=[jax.ShapeDtypeStruct((N_EDGES, H), jnp.float32),
                   jax.ShapeDtypeStruct((N_EDGES, H), jnp.float32)],
    )(geo, edge_bond_attr, xj,
      p['g1_w'], p['g1_b'][None, :], p['g2_w'], p['g2_b'][None, :],
      p['bond_w'], p['bond_b'][None, :],
      el1_w, el1_b,
      p['c1_el2_w'], p['c1_el2_b'][None, :],
      p['c2_el2_w'], p['c2_el2_b'][None, :],
      p['c1_attn_w'].T, p['c1_attn_b'][None, :],
      p['c2_attn_w'].T, p['c2_attn_b'][None, :])

    agg1 = jax.ops.segment_sum(m1, i, num_segments=N_NODES)
    agg2 = jax.ops.segment_sum(m2, i, num_segments=N_NODES)

    h, st = pl.pallas_call(
        _node_body,
        grid=(N_NODES // NB,),
        in_specs=[
            pl.BlockSpec((NB, H), lambda b: (b, 0)),
            pl.BlockSpec((NB, H), lambda b: (b, 0)),
            pl.BlockSpec((NB, H), lambda b: (b, 0)),
            _rep((H, H)), _rep((1, H)), _rep((H, H)),
            _rep((H, H)), _rep((1, H)), _rep((H, H)),
            _rep((H, H)), _rep((1, H)), _rep((H, H)), _rep((1, H)),
            _rep((2 * H, H)), _rep((1, H)),
        ],
        out_specs=[pl.BlockSpec((NB, H), lambda b: (b, 0)),
                   pl.BlockSpec((2, H), lambda b: (0, 0))],
        out_shape=[jax.ShapeDtypeStruct((N_NODES, H), jnp.float32),
                   jax.ShapeDtypeStruct((2, H), jnp.float32)],
    )(agg1, agg2, x,
      p['c1_rel_w'], p['c1_rel_b'][None, :], p['c1_root_w'],
      p['c2_rel_w'], p['c2_rel_b'][None, :], p['c2_root_w'],
      p['lin1_w'], p['lin1_b'][None, :], p['lin2_w'], p['lin2_b'][None, :],
      p['lincat_w'], p['lincat_b'][None, :])

    out = pl.pallas_call(
        _final_body,
        grid=(N_NODES // NB,),
        in_specs=[
            pl.BlockSpec((NB, H), lambda b: (b, 0)),
            _rep((2, H)),
            _rep((1, H)), _rep((1, H)),
            _rep((H, H)), _rep((1, H)), _rep((H, H)), _rep((1, H)),
        ],
        out_specs=pl.BlockSpec((NB, H), lambda b: (b, 0)),
        out_shape=jax.ShapeDtypeStruct((N_NODES, H), jnp.float32),
    )(h, st, p['gn_g'][None, :], p['gn_b'][None, :],
      p['mlp_w'], p['mlp_b'][None, :], p['final_w'], p['final_b'][None, :])

    return out


# speedup vs baseline: 1.1446x; 1.1446x over previous
import functools
import jax
import jax.numpy as jnp
from jax import lax
from jax.experimental import pallas as pl
from jax.experimental.pallas import tpu as pltpu
from jax.experimental.pallas import tpu_sc as plsc

N_NODES = 10000
N_EDGES = 160000
H = 128
MID = 64
CUTOFF = 8.0

EB = 1600
NB = 1000


def _sig(x):
    return 1.0 / (1.0 + jnp.exp(-x))


def _sw(x):
    return x * _sig(x)


def _cross(u, v):
    ux, uy, uz = u[:, 0:1], u[:, 1:2], u[:, 2:3]
    vx, vy, vz = v[:, 0:1], v[:, 1:2], v[:, 2:3]
    return jnp.concatenate(
        [uy * vz - uz * vy, uz * vx - ux * vz, ux * vy - uy * vx], axis=1)


def _dot3(u, v):
    return jnp.sum(u * v, axis=1, keepdims=True)


def _atan2pos(b, a):
    t = jnp.arctan2(b, a)
    return jnp.where(t < 0, t + jnp.pi, t)


def _edge_body(vp_ref, ga_ref, gb_ref, gc_ref, gd_ref, bond_ref, xj_ref,
               g1_w, g1_b, g2_w, g2_b, bw, bb,
               el1_w, el1_b, el2a_w, el2a_b, el2b_w, el2b_b,
               at1_w, at1_b, at2_w, at2_b,
               m1_ref, m2_ref):
    pji = vp_ref[:, 0:3]
    pin0 = ga_ref[:, 0:3]
    pin1 = gb_ref[:, 0:3]
    piref = gc_ref[:, 0:3]
    pjref = gd_ref[:, 0:3]

    npji = -pji
    a_t = _dot3(npji, pin0)
    c_t = _cross(npji, pin0)
    b_t = jnp.sqrt(_dot3(c_t, c_t) + 1e-12)
    theta = _atan2pos(b_t, a_t)

    dist = jnp.sqrt(_dot3(pji, pji) + 1e-12)

    p1 = c_t
    p2 = _cross(npji, pin1)
    a_p = _dot3(p1, p2)
    b_p = _dot3(_cross(p1, p2), pji) / dist
    phi = _atan2pos(b_p, a_p)

    q1 = _cross(pji, pjref)
    q2 = _cross(pji, piref)
    a_u = _dot3(q1, q2)
    b_u = _dot3(_cross(q1, q2), pji) / dist
    tau = _atan2pos(b_u, a_u)

    geom_in = jnp.concatenate([dist, theta, phi, tau], axis=1)
    gm = _sw(jnp.dot(geom_in, g1_w[...]) + g1_b[...])
    gm = jnp.dot(gm, g2_w[...]) + g2_b[...]
    bd = _sw(jnp.dot(bond_ref[...], bw[...]) + bb[...])

    ew = jnp.concatenate([bd, gm], axis=1)
    e1 = _sw(jnp.dot(ew, el1_w[...]) + el1_b[...])
    ew1 = jnp.dot(e1[:, :H], el2a_w[...]) + el2a_b[...]
    ew2 = jnp.dot(e1[:, H:], el2b_w[...]) + el2b_b[...]

    xj = xj_ref[...]
    m1 = ew1 * xj
    m1 = m1 * _sig(jnp.sum(m1 * at1_w[...], axis=1, keepdims=True) + at1_b[...])
    m2 = ew2 * xj
    m2 = m2 * _sig(jnp.sum(m2 * at2_w[...], axis=1, keepdims=True) + at2_b[...])
    m1_ref[...] = m1
    m2_ref[...] = m2


def _node_body(a1_ref, a2_ref, x_ref,
               rel1_w, rel1_b, root1_w, rel2_w, rel2_b, root2_w,
               lin1_w, lin1_b, lin2_w, lin2_b, cat_w, cat_b,
               h_ref, st_ref):
    x = x_ref[...]
    h1 = jnp.dot(a1_ref[...], rel1_w[...]) + rel1_b[...] + jnp.dot(x, root1_w[...])
    h1 = _sw(jnp.dot(h1, lin1_w[...]) + lin1_b[...])
    h2 = jnp.dot(a2_ref[...], rel2_w[...]) + rel2_b[...] + jnp.dot(x, root2_w[...])
    h2 = _sw(jnp.dot(h2, lin2_w[...]) + lin2_b[...])
    hc = jnp.concatenate([h1, h2], axis=1)
    h = jnp.dot(hc, cat_w[...]) + cat_b[...] + x
    h_ref[...] = h

    @pl.when(pl.program_id(0) == 0)
    def _():
        st_ref[...] = jnp.zeros_like(st_ref)

    st_ref[...] += jnp.concatenate(
        [jnp.sum(h, axis=0, keepdims=True),
         jnp.sum(h * h, axis=0, keepdims=True)], axis=0)


def _final_body(h_ref, st_ref, gn_g, gn_b, mlp_w, mlp_b, fin_w, fin_b, o_ref):
    st = st_ref[...]
    mean = st[0:1, :] / N_NODES
    var = st[1:2, :] / N_NODES - mean * mean
    h = (h_ref[...] - mean) / jnp.sqrt(var + 1e-5) * gn_g[...] + gn_b[...]
    h = _sw(jnp.dot(h, mlp_w[...]) + mlp_b[...])
    o_ref[...] = jnp.dot(h, fin_w[...]) + fin_b[...]


def _rep(shape):
    return pl.BlockSpec(shape, lambda b: (0,) * len(shape))


_NW = 32
_EPW = N_EDGES // _NW
_GB = 40
_NB_G = _EPW // _GB


def _sc_gather_body(x_hbm, jidx, xj, idxbuf, xbuf0, xbuf1, sem0, sem1):
    c = lax.axis_index("c")
    s = lax.axis_index("s")
    wid = s * 2 + c
    ebase = wid * _EPW

    pltpu.sync_copy(jidx.at[wid], idxbuf)

    pltpu.async_copy(x_hbm.at[idxbuf.at[0]], xbuf0, sem0)

    def body2(b, carry):
        even = lax.rem(b, 2) == 0
        nxt = b + 1

        @pl.when(even)
        def _():
            @pl.when(nxt < _NB_G)
            def _():
                pltpu.async_copy(x_hbm.at[idxbuf.at[nxt]], xbuf1, sem1)
            pltpu.make_async_copy(x_hbm.at[idxbuf.at[0]], xbuf0, sem0).wait()
            pltpu.sync_copy(xbuf0, xj.at[pl.ds(ebase + b * _GB, _GB)])

        @pl.when(jnp.logical_not(even))
        def _():
            @pl.when(nxt < _NB_G)
            def _():
                pltpu.async_copy(x_hbm.at[idxbuf.at[nxt]], xbuf0, sem0)
            pltpu.make_async_copy(x_hbm.at[idxbuf.at[0]], xbuf1, sem1).wait()
            pltpu.sync_copy(xbuf1, xj.at[pl.ds(ebase + b * _GB, _GB)])

        return carry

    lax.fori_loop(0, _NB_G, body2, 0)


def _sc_gather(x, jidx):
    mesh = plsc.VectorSubcoreMesh(core_axis_name="c", subcore_axis_name="s")
    f = functools.partial(
        pl.kernel, mesh=mesh,
        out_type=jax.ShapeDtypeStruct((N_EDGES, H), jnp.float32),
        scratch_types=[
            pltpu.VMEM((_NB_G, _GB), jnp.int32),
            pltpu.VMEM((_GB, H), jnp.float32),
            pltpu.VMEM((_GB, H), jnp.float32),
            pltpu.SemaphoreType.DMA,
            pltpu.SemaphoreType.DMA,
        ],
    )(_sc_gather_body)
    return f(x, jidx.reshape(_NW, _NB_G, _GB))


def _seg_min_arg(vals, seg, num_segments):
    e = vals.shape[0]
    minv = jax.ops.segment_min(vals, seg, num_segments=num_segments)
    cand = jnp.where(vals <= minv[seg], jnp.arange(e), e)
    arg = jax.ops.segment_min(cand, seg, num_segments=num_segments)
    return jnp.where(arg >= e, 0, arg)


def kernel(z, pos, edge_index, edge_bond_attr, params):
    p = params
    i = edge_index[0]
    j = edge_index[1]
    vecs = pos[j] - pos[i]
    dist = jnp.sqrt(jnp.sum(vecs * vecs, axis=-1) + 1e-12)

    seg2 = jnp.concatenate([i, j + N_NODES])
    ar = jnp.arange(N_EDGES)
    ar2 = jnp.concatenate([ar, ar])
    v2 = jnp.concatenate([dist, dist])
    minv2 = jax.ops.segment_min(v2, seg2, num_segments=2 * N_NODES)
    cand2 = jnp.where(v2 <= minv2[seg2], ar2, N_EDGES)
    arg2 = jax.ops.segment_min(cand2, seg2, num_segments=2 * N_NODES)
    arg2 = jnp.where(arg2 >= N_EDGES, 0, arg2)
    am0, am0j = arg2[:N_NODES], arg2[N_NODES:]

    add2 = jnp.zeros((2 * N_EDGES,), jnp.float32).at[
        jnp.concatenate([am0, am0j + N_EDGES])].set(CUTOFF)
    v2b = v2 + add2
    minv2b = jax.ops.segment_min(v2b, seg2, num_segments=2 * N_NODES)
    cand2b = jnp.where(v2b <= minv2b[seg2], ar2, N_EDGES)
    arg2b = jax.ops.segment_min(cand2b, seg2, num_segments=2 * N_NODES)
    arg2b = jnp.where(arg2b >= N_EDGES, 0, arg2b)
    am1, am1j = arg2b[:N_NODES], arg2b[N_NODES:]

    tbl2 = jnp.concatenate([j, i])
    nn = tbl2[jnp.concatenate([am0, am0j + N_EDGES])]
    n0, n0j = nn[:N_NODES], nn[N_NODES:]

    tbl6 = jnp.concatenate([am0, am1, n0, am0j, am1j, n0j])
    off = jnp.arange(6, dtype=i.dtype) * N_NODES
    ij6 = jnp.concatenate([i, i, i, j, j, j]) + jnp.repeat(off, N_EDGES)
    g6 = tbl6[ij6]
    am0i_e = g6[:N_EDGES]
    am1i_e = g6[N_EDGES:2 * N_EDGES]
    n0_e = g6[2 * N_EDGES:3 * N_EDGES]
    am0j_e = g6[3 * N_EDGES:4 * N_EDGES]
    am1j_e = g6[4 * N_EDGES:5 * N_EDGES]
    n0j_e = g6[5 * N_EDGES:]
    idx_iref = jnp.where(n0_e == j, am1i_e, am0i_e)
    idx_jref = jnp.where(n0j_e == i, am1j_e, am0j_e)

    x = _sw(p['emb'][z])
    vecsp = jnp.concatenate(
        [vecs, jnp.zeros((N_EDGES, 13), jnp.float32)], axis=1)
    ga = vecsp[am0i_e]
    gb = vecsp[am1i_e]
    gc = vecsp[idx_iref]
    gd = vecsp[idx_jref]
    xj = _sc_gather(x, j.astype(jnp.int32))

    el1_w = jnp.concatenate([p['c1_el1_w'], p['c2_el1_w']], axis=1)
    el1_b = jnp.concatenate([p['c1_el1_b'], p['c2_el1_b']])[None, :]

    m1, m2 = pl.pallas_call(
        _edge_body,
        grid=(N_EDGES // EB,),
        in_specs=[
            pl.BlockSpec((EB, 16), lambda b: (b, 0)),
            pl.BlockSpec((EB, 16), lambda b: (b, 0)),
            pl.BlockSpec((EB, 16), lambda b: (b, 0)),
            pl.BlockSpec((EB, 16), lambda b: (b, 0)),
            pl.BlockSpec((EB, 16), lambda b: (b, 0)),
            pl.BlockSpec((EB, 16), lambda b: (b, 0)),
            pl.BlockSpec((EB, H), lambda b: (b, 0)),
            _rep((4, MID)), _rep((1, MID)), _rep((MID, H)), _rep((1, H)),
            _rep((16, H)), _rep((1, H)),
            _rep((2 * H, 2 * H)), _rep((1, 2 * H)),
            _rep((H, H)), _rep((1, H)), _rep((H, H)), _rep((1, H)),
            _rep((1, H)), _rep((1, 1)), _rep((1, H)), _rep((1, 1)),
        ],
        out_specs=[pl.BlockSpec((EB, H), lambda b: (b, 0)),
                   pl.BlockSpec((EB, H), lambda b: (b, 0))],
        out_shape=[jax.ShapeDtypeStruct((N_EDGES, H), jnp.float32),
                   jax.ShapeDtypeStruct((N_EDGES, H), jnp.float32)],
    )(vecsp, ga, gb, gc, gd, edge_bond_attr, xj,
      p['g1_w'], p['g1_b'][None, :], p['g2_w'], p['g2_b'][None, :],
      p['bond_w'], p['bond_b'][None, :],
      el1_w, el1_b,
      p['c1_el2_w'], p['c1_el2_b'][None, :],
      p['c2_el2_w'], p['c2_el2_b'][None, :],
      p['c1_attn_w'].T, p['c1_attn_b'][None, :],
      p['c2_attn_w'].T, p['c2_attn_b'][None, :])

    agg1 = jax.ops.segment_sum(m1, i, num_segments=N_NODES)
    agg2 = jax.ops.segment_sum(m2, i, num_segments=N_NODES)

    h, st = pl.pallas_call(
        _node_body,
        grid=(N_NODES // NB,),
        in_specs=[
            pl.BlockSpec((NB, H), lambda b: (b, 0)),
            pl.BlockSpec((NB, H), lambda b: (b, 0)),
            pl.BlockSpec((NB, H), lambda b: (b, 0)),
            _rep((H, H)), _rep((1, H)), _rep((H, H)),
            _rep((H, H)), _rep((1, H)), _rep((H, H)),
            _rep((H, H)), _rep((1, H)), _rep((H, H)), _rep((1, H)),
            _rep((2 * H, H)), _rep((1, H)),
        ],
        out_specs=[pl.BlockSpec((NB, H), lambda b: (b, 0)),
                   pl.BlockSpec((2, H), lambda b: (0, 0))],
        out_shape=[jax.ShapeDtypeStruct((N_NODES, H), jnp.float32),
                   jax.ShapeDtypeStruct((2, H), jnp.float32)],
    )(agg1, agg2, x,
      p['c1_rel_w'], p['c1_rel_b'][None, :], p['c1_root_w'],
      p['c2_rel_w'], p['c2_rel_b'][None, :], p['c2_root_w'],
      p['lin1_w'], p['lin1_b'][None, :], p['lin2_w'], p['lin2_b'][None, :],
      p['lincat_w'], p['lincat_b'][None, :])

    out = pl.pallas_call(
        _final_body,
        grid=(N_NODES // NB,),
        in_specs=[
            pl.BlockSpec((NB, H), lambda b: (b, 0)),
            _rep((2, H)),
            _rep((1, H)), _rep((1, H)),
            _rep((H, H)), _rep((1, H)), _rep((H, H)), _rep((1, H)),
        ],
        out_specs=pl.BlockSpec((NB, H), lambda b: (b, 0)),
        out_shape=jax.ShapeDtypeStruct((N_NODES, H), jnp.float32),
    )(h, st, p['gn_g'][None, :], p['gn_b'][None, :],
      p['mlp_w'], p['mlp_b'][None, :], p['final_w'], p['final_b'][None, :])

    return out


# speedup vs baseline: 1.1829x; 1.0334x over previous
import functools
import jax
import jax.numpy as jnp
from jax import lax
from jax.experimental import pallas as pl
from jax.experimental.pallas import tpu as pltpu
from jax.experimental.pallas import tpu_sc as plsc

N_NODES = 10000
N_EDGES = 160000
H = 128
MID = 64
CUTOFF = 8.0

EB = 1600
NB = 1000


def _sig(x):
    return 1.0 / (1.0 + jnp.exp(-x))


def _sw(x):
    return x * _sig(x)


def _cross(u, v):
    ux, uy, uz = u[:, 0:1], u[:, 1:2], u[:, 2:3]
    vx, vy, vz = v[:, 0:1], v[:, 1:2], v[:, 2:3]
    return jnp.concatenate(
        [uy * vz - uz * vy, uz * vx - ux * vz, ux * vy - uy * vx], axis=1)


def _dot3(u, v):
    return jnp.sum(u * v, axis=1, keepdims=True)


def _atan2pos(b, a):
    t = jnp.arctan2(b, a)
    return jnp.where(t < 0, t + jnp.pi, t)


def _edge_body(vp_ref, ga_ref, gb_ref, gc_ref, gd_ref, bond_ref, xj_ref,
               g1_w, g1_b, g2_w, g2_b, bw, bb,
               el1_w, el1_b, el2a_w, el2a_b, el2b_w, el2b_b,
               at1_w, at1_b, at2_w, at2_b,
               m1_ref, m2_ref):
    pji = vp_ref[:, 0:3]
    pin0 = ga_ref[:, 0:3]
    pin1 = gb_ref[:, 0:3]
    piref = gc_ref[:, 0:3]
    pjref = gd_ref[:, 0:3]

    npji = -pji
    a_t = _dot3(npji, pin0)
    c_t = _cross(npji, pin0)
    b_t = jnp.sqrt(_dot3(c_t, c_t) + 1e-12)
    theta = _atan2pos(b_t, a_t)

    dist = jnp.sqrt(_dot3(pji, pji) + 1e-12)

    p1 = c_t
    p2 = _cross(npji, pin1)
    a_p = _dot3(p1, p2)
    b_p = _dot3(_cross(p1, p2), pji) / dist
    phi = _atan2pos(b_p, a_p)

    q1 = _cross(pji, pjref)
    q2 = _cross(pji, piref)
    a_u = _dot3(q1, q2)
    b_u = _dot3(_cross(q1, q2), pji) / dist
    tau = _atan2pos(b_u, a_u)

    geom_in = jnp.concatenate([dist, theta, phi, tau], axis=1)
    gm = _sw(jnp.dot(geom_in, g1_w[...]) + g1_b[...])
    gm = jnp.dot(gm, g2_w[...]) + g2_b[...]
    bd = _sw(jnp.dot(bond_ref[...], bw[...]) + bb[...])

    ew = jnp.concatenate([bd, gm], axis=1)
    e1 = _sw(jnp.dot(ew, el1_w[...]) + el1_b[...])
    ew1 = jnp.dot(e1[:, :H], el2a_w[...]) + el2a_b[...]
    ew2 = jnp.dot(e1[:, H:], el2b_w[...]) + el2b_b[...]

    xj = xj_ref[...]
    m1 = ew1 * xj
    m1 = m1 * _sig(jnp.sum(m1 * at1_w[...], axis=1, keepdims=True) + at1_b[...])
    m2 = ew2 * xj
    m2 = m2 * _sig(jnp.sum(m2 * at2_w[...], axis=1, keepdims=True) + at2_b[...])
    m1_ref[...] = m1
    m2_ref[...] = m2


def _node_body(a1_ref, a2_ref, x_ref,
               rel1_w, rel1_b, root1_w, rel2_w, rel2_b, root2_w,
               lin1_w, lin1_b, lin2_w, lin2_b, cat_w, cat_b,
               h_ref, st_ref):
    x = x_ref[...]
    h1 = jnp.dot(a1_ref[...], rel1_w[...]) + rel1_b[...] + jnp.dot(x, root1_w[...])
    h1 = _sw(jnp.dot(h1, lin1_w[...]) + lin1_b[...])
    h2 = jnp.dot(a2_ref[...], rel2_w[...]) + rel2_b[...] + jnp.dot(x, root2_w[...])
    h2 = _sw(jnp.dot(h2, lin2_w[...]) + lin2_b[...])
    hc = jnp.concatenate([h1, h2], axis=1)
    h = jnp.dot(hc, cat_w[...]) + cat_b[...] + x
    h_ref[...] = h

    @pl.when(pl.program_id(0) == 0)
    def _():
        st_ref[...] = jnp.zeros_like(st_ref)

    st_ref[...] += jnp.concatenate(
        [jnp.sum(h, axis=0, keepdims=True),
         jnp.sum(h * h, axis=0, keepdims=True)], axis=0)


def _final_body(h_ref, st_ref, gn_g, gn_b, mlp_w, mlp_b, fin_w, fin_b, o_ref):
    st = st_ref[...]
    mean = st[0:1, :] / N_NODES
    var = st[1:2, :] / N_NODES - mean * mean
    h = (h_ref[...] - mean) / jnp.sqrt(var + 1e-5) * gn_g[...] + gn_b[...]
    h = _sw(jnp.dot(h, mlp_w[...]) + mlp_b[...])
    o_ref[...] = jnp.dot(h, fin_w[...]) + fin_b[...]


def _rep(shape):
    return pl.BlockSpec(shape, lambda b: (0,) * len(shape))


_NW = 32
_EPW = N_EDGES // _NW
_GB = 40
_NB_G = _EPW // _GB


def _sc_gather_body(x_hbm, jidx, xj, idxbuf, xbuf0, xbuf1, sem0, sem1):
    c = lax.axis_index("c")
    s = lax.axis_index("s")
    wid = s * 2 + c
    ebase = wid * _EPW

    pltpu.sync_copy(jidx.at[wid], idxbuf)

    pltpu.async_copy(x_hbm.at[idxbuf.at[0]], xbuf0, sem0)

    def body2(b, carry):
        even = lax.rem(b, 2) == 0
        nxt = b + 1

        @pl.when(even)
        def _():
            @pl.when(nxt < _NB_G)
            def _():
                pltpu.async_copy(x_hbm.at[idxbuf.at[nxt]], xbuf1, sem1)
            pltpu.make_async_copy(x_hbm.at[idxbuf.at[0]], xbuf0, sem0).wait()
            pltpu.sync_copy(xbuf0, xj.at[pl.ds(ebase + b * _GB, _GB)])

        @pl.when(jnp.logical_not(even))
        def _():
            @pl.when(nxt < _NB_G)
            def _():
                pltpu.async_copy(x_hbm.at[idxbuf.at[nxt]], xbuf0, sem0)
            pltpu.make_async_copy(x_hbm.at[idxbuf.at[0]], xbuf1, sem1).wait()
            pltpu.sync_copy(xbuf1, xj.at[pl.ds(ebase + b * _GB, _GB)])

        return carry

    lax.fori_loop(0, _NB_G, body2, 0)


def _sc_gather(x, jidx):
    mesh = plsc.VectorSubcoreMesh(core_axis_name="c", subcore_axis_name="s")
    f = functools.partial(
        pl.kernel, mesh=mesh,
        out_type=jax.ShapeDtypeStruct((N_EDGES, H), jnp.float32),
        scratch_types=[
            pltpu.VMEM((_NB_G, _GB), jnp.int32),
            pltpu.VMEM((_GB, H), jnp.float32),
            pltpu.VMEM((_GB, H), jnp.float32),
            pltpu.SemaphoreType.DMA,
            pltpu.SemaphoreType.DMA,
        ],
    )(_sc_gather_body)
    return f(x, jidx.reshape(_NW, _NB_G, _GB))


_ZCH = 624


def _sc_segsum_body(m1, m2, iidx, zrows, out, idxbuf, mbuf, acc):
    c = lax.axis_index("c")
    s = lax.axis_index("s")
    wid = s * 2 + c
    ebase = wid * _EPW
    zbase = s * _ZCH
    rem = N_NODES - 16 * _ZCH

    pltpu.sync_copy(iidx.at[wid], idxbuf)

    for conv in range(2):
        m_hbm = (m1, m2)[conv]
        pltpu.sync_copy(zrows.at[pl.ds(0, _ZCH)], acc.at[pl.ds(zbase, _ZCH)])

        @pl.when(s == 15)
        def _():
            pltpu.sync_copy(zrows.at[pl.ds(0, rem)],
                            acc.at[pl.ds(16 * _ZCH, rem)])

        plsc.subcore_barrier()

        def body(b, carry):
            pltpu.sync_copy(m_hbm.at[pl.ds(ebase + b * _GB, _GB)], mbuf)
            pltpu.sync_copy(mbuf, acc.at[idxbuf.at[b]], add=True)
            return carry

        lax.fori_loop(0, _NB_G, body, 0)
        plsc.subcore_barrier()
        pltpu.sync_copy(acc.at[pl.ds(zbase, _ZCH)],
                        out.at[conv, c, pl.ds(zbase, _ZCH)])

        @pl.when(s == 15)
        def _():
            pltpu.sync_copy(acc.at[pl.ds(16 * _ZCH, rem)],
                            out.at[conv, c, pl.ds(16 * _ZCH, rem)])

        plsc.subcore_barrier()


def _sc_segsum(m1, m2, iidx):
    mesh = plsc.VectorSubcoreMesh(core_axis_name="c", subcore_axis_name="s")
    zrows = jnp.zeros((640, H), jnp.float32)
    f = functools.partial(
        pl.kernel, mesh=mesh,
        out_type=jax.ShapeDtypeStruct((2, 2, N_NODES, H), jnp.float32),
        scratch_types=[
            pltpu.VMEM((_NB_G, _GB), jnp.int32),
            pltpu.VMEM((_GB, H), jnp.float32),
            pltpu.VMEM_SHARED((N_NODES, H), jnp.float32),
        ],
    )(_sc_segsum_body)
    parts = f(m1, m2, iidx.reshape(_NW, _NB_G, _GB), zrows)
    return parts[0, 0] + parts[0, 1], parts[1, 0] + parts[1, 1]


def _seg_min_arg(vals, seg, num_segments):
    e = vals.shape[0]
    minv = jax.ops.segment_min(vals, seg, num_segments=num_segments)
    cand = jnp.where(vals <= minv[seg], jnp.arange(e), e)
    arg = jax.ops.segment_min(cand, seg, num_segments=num_segments)
    return jnp.where(arg >= e, 0, arg)


def kernel(z, pos, edge_index, edge_bond_attr, params):
    p = params
    i = edge_index[0]
    j = edge_index[1]
    vecs = pos[j] - pos[i]
    dist = jnp.sqrt(jnp.sum(vecs * vecs, axis=-1) + 1e-12)

    seg2 = jnp.concatenate([i, j + N_NODES])
    ar = jnp.arange(N_EDGES)
    ar2 = jnp.concatenate([ar, ar])
    v2 = jnp.concatenate([dist, dist])
    minv2 = jax.ops.segment_min(v2, seg2, num_segments=2 * N_NODES)
    cand2 = jnp.where(v2 <= minv2[seg2], ar2, N_EDGES)
    arg2 = jax.ops.segment_min(cand2, seg2, num_segments=2 * N_NODES)
    arg2 = jnp.where(arg2 >= N_EDGES, 0, arg2)
    am0, am0j = arg2[:N_NODES], arg2[N_NODES:]

    add2 = jnp.zeros((2 * N_EDGES,), jnp.float32).at[
        jnp.concatenate([am0, am0j + N_EDGES])].set(CUTOFF)
    v2b = v2 + add2
    minv2b = jax.ops.segment_min(v2b, seg2, num_segments=2 * N_NODES)
    cand2b = jnp.where(v2b <= minv2b[seg2], ar2, N_EDGES)
    arg2b = jax.ops.segment_min(cand2b, seg2, num_segments=2 * N_NODES)
    arg2b = jnp.where(arg2b >= N_EDGES, 0, arg2b)
    am1, am1j = arg2b[:N_NODES], arg2b[N_NODES:]

    tbl2 = jnp.concatenate([j, i])
    nn = tbl2[jnp.concatenate([am0, am0j + N_EDGES])]
    n0, n0j = nn[:N_NODES], nn[N_NODES:]

    tbl6 = jnp.concatenate([am0, am1, n0, am0j, am1j, n0j])
    off = jnp.arange(6, dtype=i.dtype) * N_NODES
    ij6 = jnp.concatenate([i, i, i, j, j, j]) + jnp.repeat(off, N_EDGES)
    g6 = tbl6[ij6]
    am0i_e = g6[:N_EDGES]
    am1i_e = g6[N_EDGES:2 * N_EDGES]
    n0_e = g6[2 * N_EDGES:3 * N_EDGES]
    am0j_e = g6[3 * N_EDGES:4 * N_EDGES]
    am1j_e = g6[4 * N_EDGES:5 * N_EDGES]
    n0j_e = g6[5 * N_EDGES:]
    idx_iref = jnp.where(n0_e == j, am1i_e, am0i_e)
    idx_jref = jnp.where(n0j_e == i, am1j_e, am0j_e)

    x = _sw(p['emb'][z])
    vecsp = jnp.concatenate(
        [vecs, jnp.zeros((N_EDGES, 13), jnp.float32)], axis=1)
    ga = vecsp[am0i_e]
    gb = vecsp[am1i_e]
    gc = vecsp[idx_iref]
    gd = vecsp[idx_jref]
    xj = _sc_gather(x, j.astype(jnp.int32))

    el1_w = jnp.concatenate([p['c1_el1_w'], p['c2_el1_w']], axis=1)
    el1_b = jnp.concatenate([p['c1_el1_b'], p['c2_el1_b']])[None, :]

    m1, m2 = pl.pallas_call(
        _edge_body,
        grid=(N_EDGES // EB,),
        in_specs=[
            pl.BlockSpec((EB, 16), lambda b: (b, 0)),
            pl.BlockSpec((EB, 16), lambda b: (b, 0)),
            pl.BlockSpec((EB, 16), lambda b: (b, 0)),
            pl.BlockSpec((EB, 16), lambda b: (b, 0)),
            pl.BlockSpec((EB, 16), lambda b: (b, 0)),
            pl.BlockSpec((EB, 16), lambda b: (b, 0)),
            pl.BlockSpec((EB, H), lambda b: (b, 0)),
            _rep((4, MID)), _rep((1, MID)), _rep((MID, H)), _rep((1, H)),
            _rep((16, H)), _rep((1, H)),
            _rep((2 * H, 2 * H)), _rep((1, 2 * H)),
            _rep((H, H)), _rep((1, H)), _rep((H, H)), _rep((1, H)),
            _rep((1, H)), _rep((1, 1)), _rep((1, H)), _rep((1, 1)),
        ],
        out_specs=[pl.BlockSpec((EB, H), lambda b: (b, 0)),
                   pl.BlockSpec((EB, H), lambda b: (b, 0))],
        out_shape=[jax.ShapeDtypeStruct((N_EDGES, H), jnp.float32),
                   jax.ShapeDtypeStruct((N_EDGES, H), jnp.float32)],
    )(vecsp, ga, gb, gc, gd, edge_bond_attr, xj,
      p['g1_w'], p['g1_b'][None, :], p['g2_w'], p['g2_b'][None, :],
      p['bond_w'], p['bond_b'][None, :],
      el1_w, el1_b,
      p['c1_el2_w'], p['c1_el2_b'][None, :],
      p['c2_el2_w'], p['c2_el2_b'][None, :],
      p['c1_attn_w'].T, p['c1_attn_b'][None, :],
      p['c2_attn_w'].T, p['c2_attn_b'][None, :])

    agg1, agg2 = _sc_segsum(m1, m2, i.astype(jnp.int32))

    h, st = pl.pallas_call(
        _node_body,
        grid=(N_NODES // NB,),
        in_specs=[
            pl.BlockSpec((NB, H), lambda b: (b, 0)),
            pl.BlockSpec((NB, H), lambda b: (b, 0)),
            pl.BlockSpec((NB, H), lambda b: (b, 0)),
            _rep((H, H)), _rep((1, H)), _rep((H, H)),
            _rep((H, H)), _rep((1, H)), _rep((H, H)),
            _rep((H, H)), _rep((1, H)), _rep((H, H)), _rep((1, H)),
            _rep((2 * H, H)), _rep((1, H)),
        ],
        out_specs=[pl.BlockSpec((NB, H), lambda b: (b, 0)),
                   pl.BlockSpec((2, H), lambda b: (0, 0))],
        out_shape=[jax.ShapeDtypeStruct((N_NODES, H), jnp.float32),
                   jax.ShapeDtypeStruct((2, H), jnp.float32)],
    )(agg1, agg2, x,
      p['c1_rel_w'], p['c1_rel_b'][None, :], p['c1_root_w'],
      p['c2_rel_w'], p['c2_rel_b'][None, :], p['c2_root_w'],
      p['lin1_w'], p['lin1_b'][None, :], p['lin2_w'], p['lin2_b'][None, :],
      p['lincat_w'], p['lincat_b'][None, :])

    out = pl.pallas_call(
        _final_body,
        grid=(N_NODES // NB,),
        in_specs=[
            pl.BlockSpec((NB, H), lambda b: (b, 0)),
            _rep((2, H)),
            _rep((1, H)), _rep((1, H)),
            _rep((H, H)), _rep((1, H)), _rep((H, H)), _rep((1, H)),
        ],
        out_specs=pl.BlockSpec((NB, H), lambda b: (b, 0)),
        out_shape=jax.ShapeDtypeStruct((N_NODES, H), jnp.float32),
    )(h, st, p['gn_g'][None, :], p['gn_b'][None, :],
      p['mlp_w'], p['mlp_b'][None, :], p['final_w'], p['final_b'][None, :])

    return out
